# baseline (speedup 1.0000x reference)
import functools

import jax
import jax.numpy as jnp
import numpy as np
from jax import lax
from jax.experimental import pallas as pl
from jax.experimental.pallas import tpu as pltpu
from jax.experimental.pallas import tpu_sc as plsc

_SIZE = 32000
_N = 2048
_PAD = 0
_SMOOTH = 0.1
_CONF = 1.0 - _SMOOTH
_EPS = _SMOOTH / (_SIZE - 2)
_C1 = np.float32(_EPS * np.log(_EPS) * (_SIZE - 2) + _CONF * np.log(_CONF))

_BR = 256
_BC = 16000


_NC = 2
_NS = 16
_NW = _NC * _NS
_BPW = _N // _NW


@functools.partial(
    pl.kernel,
    out_type=jax.ShapeDtypeStruct((_N,), jnp.float32),
    mesh=plsc.VectorSubcoreMesh(core_axis_name="c", subcore_axis_name="s"),
    scratch_types=[
        pltpu.VMEM((_BPW,), jnp.int32),
        pltpu.VMEM((_BPW,), jnp.int32),
        pltpu.VMEM((_BPW,), jnp.float32),
        pltpu.SemaphoreType.DMA,
    ],
)
def _sc_gather(x_hbm, tgt_hbm, out_hbm, tgt_v, idx_v, val_v, sem):
    wid = lax.axis_index("s") * _NC + lax.axis_index("c")
    base = wid * _BPW
    pltpu.sync_copy(tgt_hbm.at[pl.ds(base, _BPW)], tgt_v)
    for v in range(_BPW // 16):
        rows = base + v * 16 + lax.iota(jnp.int32, 16)
        sl = pl.ds(v * 16, 16)
        idx_v[sl] = rows * _SIZE + tgt_v[sl]
    pltpu.async_copy(x_hbm.at[idx_v], val_v, sem).wait()
    pltpu.sync_copy(val_v, out_hbm.at[pl.ds(base, _BPW)])


def _loss_block(x_ref, t_ref, p_ref, kl_ref, nll_ref):
    i = pl.program_id(0)
    j = pl.program_id(1)
    xb = x_ref[...]
    tb = t_ref[...]
    maskf = (tb != _PAD).astype(jnp.float32)

    rowsum = jnp.sum(xb, axis=1, keepdims=True)
    msum = jnp.sum(rowsum * maskf)

    first = (i == 0) & (j == 0)

    @pl.when(first)
    def _():
        kl_ref[...] = jnp.zeros((1, 1), jnp.float32)
        nll_ref[...] = jnp.zeros((1, 1), jnp.float32)

    @pl.when(j == 0)
    def _():
        g = jnp.sum(p_ref[...] * maskf)
        extra = (_EPS * jnp.sum(xb[:, 0:1] * maskf)
                 + _C1 * jnp.sum(maskf) + (_EPS - _CONF) * g)
        kl_ref[...] += extra.reshape(1, 1)
        nll_ref[...] += (-g).reshape(1, 1)

    kl_ref[...] += (-_EPS * msum).reshape(1, 1)


@jax.jit
def kernel(x, target):
    t32 = target.astype(jnp.int32)
    picked = _sc_gather(x.reshape(-1), t32)
    t2d = t32.reshape(_N, 1)
    p2d = picked.reshape(_N, 1)
    kl, nll = pl.pallas_call(
        _loss_block,
        grid=(_N // _BR, _SIZE // _BC),
        in_specs=[
            pl.BlockSpec((_BR, _BC), lambda i, j: (i, j)),
            pl.BlockSpec((_BR, 1), lambda i, j: (i, 0)),
            pl.BlockSpec((_BR, 1), lambda i, j: (i, 0)),
        ],
        out_specs=[
            pl.BlockSpec((1, 1), lambda i, j: (0, 0)),
            pl.BlockSpec((1, 1), lambda i, j: (0, 0)),
        ],
        out_shape=[
            jax.ShapeDtypeStruct((1, 1), jnp.float32),
            jax.ShapeDtypeStruct((1, 1), jnp.float32),
        ],
    )(x, t2d, p2d)
    return (kl[0, 0], nll[0, 0])


# speedup vs baseline: 2.2855x; 2.2855x over previous
import functools

import jax
import jax.numpy as jnp
import numpy as np
from jax import lax
from jax.experimental import pallas as pl
from jax.experimental.pallas import tpu as pltpu
from jax.experimental.pallas import tpu_sc as plsc

_SIZE = 32000
_N = 2048
_PAD = 0
_SMOOTH = 0.1
_CONF = 1.0 - _SMOOTH
_EPS = _SMOOTH / (_SIZE - 2)
_C1 = np.float32(_EPS * np.log(_EPS) * (_SIZE - 2) + _CONF * np.log(_CONF))

_CT = 22400
_BR = 256
_BC = 4480

_SC_CHUNK = 3200
_NC = 2
_NS = 16
_NW = _NC * _NS
_BPW = _N // _NW
_NCHUNK = (_SIZE - _CT) // _SC_CHUNK


def _bcast_lane(vec, lane):
    idx = jnp.full((16, 1), lane, jnp.int32)
    dn = lax.GatherDimensionNumbers(
        offset_dims=(), collapsed_slice_dims=(0,), start_index_map=(0,))
    return lax.gather(vec, idx, dn, (1,),
                      mode=lax.GatherScatterMode.PROMISE_IN_BOUNDS)


@functools.partial(
    pl.kernel,
    out_type=[
        jax.ShapeDtypeStruct((_N * 16,), jnp.float32),
        jax.ShapeDtypeStruct((_N * 16,), jnp.float32),
    ],
    mesh=plsc.VectorSubcoreMesh(core_axis_name="c", subcore_axis_name="s"),
    scratch_types=[
        pltpu.VMEM((_BPW,), jnp.int32),
        pltpu.VMEM((8, _SC_CHUNK), jnp.float32),
        pltpu.VMEM((_BPW * 16,), jnp.float32),
        pltpu.VMEM((_BPW * 16,), jnp.float32),
        pltpu.SemaphoreType.DMA,
    ],
)
def _sc_colsum(x_hbm, tgt_hbm, rs_hbm, pk_hbm, tgt_v, buf_v, rs_v, pk_v, sem):
    cid = lax.axis_index("c")
    sid = lax.axis_index("s")
    wid = sid * _NC + cid
    base = wid * _BPW
    lanes = lax.iota(jnp.int32, 16)
    pltpu.sync_copy(tgt_hbm.at[pl.ds(base, _BPW)], tgt_v)
    for g in range(_BPW // 8):
        row0 = base + g * 8
        tvec = tgt_v[pl.ds((g // 2) * 16, 16)]
        tb = [_bcast_lane(tvec, 8 * (g % 2) + r) for r in range(8)]
        acc_rs = [jnp.zeros((16,), jnp.float32) for _ in range(8)]
        acc_pk = [jnp.zeros((16,), jnp.float32) for _ in range(8)]
        for ch in range(_NCHUNK):
            col0 = _CT + ch * _SC_CHUNK
            pltpu.sync_copy(
                x_hbm.at[pl.ds(row0, 8), pl.ds(col0, _SC_CHUNK)], buf_v)

            def body(i, carry, col0=col0, tb=tb):
                rs, pk = carry
                cvec = col0 + i * 16 + lanes
                rs2, pk2 = [], []
                for r in range(8):
                    v = buf_v[r, pl.ds(i * 16, 16)]
                    rs2.append(rs[r] + v)
                    pk2.append(pk[r] + jnp.where(cvec == tb[r], v, 0.0))
                return rs2, pk2

            acc_rs, acc_pk = lax.fori_loop(
                0, _SC_CHUNK // 16, body, (acc_rs, acc_pk))
        for r in range(8):
            sl = pl.ds((g * 8 + r) * 16, 16)
            rs_v[sl] = acc_rs[r]
            pk_v[sl] = acc_pk[r]
    pltpu.sync_copy(rs_v, rs_hbm.at[pl.ds(base * 16, _BPW * 16)])
    pltpu.sync_copy(pk_v, pk_hbm.at[pl.ds(base * 16, _BPW * 16)])


def _loss_block(x_ref, t_ref, kl_ref, nll_ref):
    i = pl.program_id(0)
    j = pl.program_id(1)
    xb = x_ref[...]
    tb = t_ref[...]
    maskf = (tb != _PAD).astype(jnp.float32)

    rowsum = jnp.sum(xb, axis=1, keepdims=True)
    colid = j * _BC + lax.broadcasted_iota(jnp.int32, (_BR, _BC), 1)
    rowg = jnp.sum(jnp.where(colid == tb, xb, 0.0), axis=1, keepdims=True)

    msum = jnp.sum(rowsum * maskf)
    g = jnp.sum(rowg * maskf)

    @pl.when((i == 0) & (j == 0))
    def _():
        kl_ref[...] = jnp.zeros((1, 1), jnp.float32)
        nll_ref[...] = jnp.zeros((1, 1), jnp.float32)

    @pl.when(j == 0)
    def _():
        extra = _EPS * jnp.sum(xb[:, 0:1] * maskf) + _C1 * jnp.sum(maskf)
        kl_ref[...] += extra.reshape(1, 1)

    kl_ref[...] += (-_EPS * msum + (_EPS - _CONF) * g).reshape(1, 1)
    nll_ref[...] += (-g).reshape(1, 1)


def _sc_combine(rs_ref, pk_ref, t_ref, kl_ref, nll_ref):
    maskf = (t_ref[...] != _PAD).astype(jnp.float32)
    rs_row = jnp.sum(rs_ref[...], axis=1, keepdims=True)
    pk_row = jnp.sum(pk_ref[...], axis=1, keepdims=True)
    msum = jnp.sum(rs_row * maskf)
    g = jnp.sum(pk_row * maskf)
    kl_ref[...] = (-_EPS * msum + (_EPS - _CONF) * g).reshape(1, 1)
    nll_ref[...] = (-g).reshape(1, 1)


@jax.jit
def kernel(x, target):
    t32 = target.astype(jnp.int32)
    rs_flat, pk_flat = _sc_colsum(x, t32)
    t2d = t32.reshape(_N, 1)
    kl_tc, nll_tc = pl.pallas_call(
        _loss_block,
        grid=(_N // _BR, _CT // _BC),
        in_specs=[
            pl.BlockSpec((_BR, _BC), lambda i, j: (i, j)),
            pl.BlockSpec((_BR, 1), lambda i, j: (i, 0)),
        ],
        out_specs=[
            pl.BlockSpec((1, 1), lambda i, j: (0, 0)),
            pl.BlockSpec((1, 1), lambda i, j: (0, 0)),
        ],
        out_shape=[
            jax.ShapeDtypeStruct((1, 1), jnp.float32),
            jax.ShapeDtypeStruct((1, 1), jnp.float32),
        ],
    )(x, t2d)
    kl_sc, nll_sc = pl.pallas_call(
        _sc_combine,
        out_shape=[
            jax.ShapeDtypeStruct((1, 1), jnp.float32),
            jax.ShapeDtypeStruct((1, 1), jnp.float32),
        ],
    )(rs_flat.reshape(_N, 16), pk_flat.reshape(_N, 16), t2d)
    kl = kl_tc[0, 0] + kl_sc[0, 0]
    nll = nll_tc[0, 0] + nll_sc[0, 0]
    return (kl, nll)


# speedup vs baseline: 2.3711x; 1.0374x over previous
import functools

import jax
import jax.numpy as jnp
import numpy as np
from jax import lax
from jax.experimental import pallas as pl
from jax.experimental.pallas import tpu as pltpu
from jax.experimental.pallas import tpu_sc as plsc

_SIZE = 32000
_N = 2048
_PAD = 0
_SMOOTH = 0.1
_CONF = 1.0 - _SMOOTH
_EPS = _SMOOTH / (_SIZE - 2)
_C1 = np.float32(_EPS * np.log(_EPS) * (_SIZE - 2) + _CONF * np.log(_CONF))

_CT = 22400
_BR = 256
_BC = 4480

_SC_CHUNK = 3200
_NC = 2
_NS = 16
_NW = _NC * _NS
_BPW = _N // _NW
_NCHUNK = (_SIZE - _CT) // _SC_CHUNK


def _bcast_lane(vec, lane):
    idx = jnp.full((16, 1), lane, jnp.int32)
    dn = lax.GatherDimensionNumbers(
        offset_dims=(), collapsed_slice_dims=(0,), start_index_map=(0,))
    return lax.gather(vec, idx, dn, (1,),
                      mode=lax.GatherScatterMode.PROMISE_IN_BOUNDS)


@functools.partial(
    pl.kernel,
    out_type=[
        jax.ShapeDtypeStruct((_N * 16,), jnp.float32),
        jax.ShapeDtypeStruct((_N * 16,), jnp.float32),
    ],
    mesh=plsc.VectorSubcoreMesh(core_axis_name="c", subcore_axis_name="s"),
    scratch_types=[
        pltpu.VMEM((_BPW,), jnp.int32),
        pltpu.VMEM((8, _SC_CHUNK), jnp.float32),
        pltpu.VMEM((8, _SC_CHUNK), jnp.float32),
        pltpu.VMEM((_BPW * 16,), jnp.float32),
        pltpu.VMEM((_BPW * 16,), jnp.float32),
        pltpu.SemaphoreType.DMA,
    ],
)
def _sc_colsum(x_hbm, tgt_hbm, rs_hbm, pk_hbm, tgt_v, buf0_v, buf1_v,
               rs_v, pk_v, sem):
    cid = lax.axis_index("c")
    sid = lax.axis_index("s")
    wid = sid * _NC + cid
    base = wid * _BPW
    lanes = lax.iota(jnp.int32, 16)
    pltpu.sync_copy(tgt_hbm.at[pl.ds(base, _BPW)], tgt_v)
    bufs = [buf0_v, buf1_v]
    ngroup = _BPW // 8
    steps = [(g, ch) for g in range(ngroup) for ch in range(_NCHUNK)]

    def start(step):
        g, ch = steps[step]
        return pltpu.async_copy(
            x_hbm.at[pl.ds(base + g * 8, 8),
                     pl.ds(_CT + ch * _SC_CHUNK, _SC_CHUNK)],
            bufs[step % 2], sem)

    pending = start(0)
    acc_rs = [jnp.zeros((16,), jnp.float32) for _ in range(8)]
    acc_pk = [jnp.zeros((16,), jnp.float32) for _ in range(8)]
    for step, (g, ch) in enumerate(steps):
        pending.wait()
        if step + 1 < len(steps):
            pending = start(step + 1)
        buf_v = bufs[step % 2]
        col0 = _CT + ch * _SC_CHUNK
        tvec = tgt_v[pl.ds((g // 2) * 16, 16)]
        tb = [_bcast_lane(tvec, 8 * (g % 2) + r) for r in range(8)]

        def body(i, carry, col0=col0, tb=tb, buf_v=buf_v):
            rs, pk = carry
            cvec = col0 + i * 16 + lanes
            rs2, pk2 = [], []
            for r in range(8):
                v = buf_v[r, pl.ds(i * 16, 16)]
                rs2.append(rs[r] + v)
                pk2.append(pk[r] + jnp.where(cvec == tb[r], v, 0.0))
            return rs2, pk2

        acc_rs, acc_pk = lax.fori_loop(
            0, _SC_CHUNK // 16, body, (acc_rs, acc_pk))
        if ch == _NCHUNK - 1:
            for r in range(8):
                sl = pl.ds((g * 8 + r) * 16, 16)
                rs_v[sl] = acc_rs[r]
                pk_v[sl] = acc_pk[r]
            acc_rs = [jnp.zeros((16,), jnp.float32) for _ in range(8)]
            acc_pk = [jnp.zeros((16,), jnp.float32) for _ in range(8)]
    pltpu.sync_copy(rs_v, rs_hbm.at[pl.ds(base * 16, _BPW * 16)])
    pltpu.sync_copy(pk_v, pk_hbm.at[pl.ds(base * 16, _BPW * 16)])


def _loss_block(x_ref, t_ref, kl_ref, nll_ref):
    i = pl.program_id(0)
    j = pl.program_id(1)
    xb = x_ref[...]
    tb = t_ref[...]
    maskf = (tb != _PAD).astype(jnp.float32)

    rowsum = jnp.sum(xb, axis=1, keepdims=True)
    colid = j * _BC + lax.broadcasted_iota(jnp.int32, (_BR, _BC), 1)
    rowg = jnp.sum(jnp.where(colid == tb, xb, 0.0), axis=1, keepdims=True)

    msum = jnp.sum(rowsum * maskf)
    g = jnp.sum(rowg * maskf)

    @pl.when((i == 0) & (j == 0))
    def _():
        kl_ref[...] = jnp.zeros((1, 1), jnp.float32)
        nll_ref[...] = jnp.zeros((1, 1), jnp.float32)

    @pl.when(j == 0)
    def _():
        extra = _EPS * jnp.sum(xb[:, 0:1] * maskf) + _C1 * jnp.sum(maskf)
        kl_ref[...] += extra.reshape(1, 1)

    kl_ref[...] += (-_EPS * msum + (_EPS - _CONF) * g).reshape(1, 1)
    nll_ref[...] += (-g).reshape(1, 1)


def _sc_combine(rs_ref, pk_ref, t_ref, kl_ref, nll_ref):
    maskf = (t_ref[...] != _PAD).astype(jnp.float32)
    rs_row = jnp.sum(rs_ref[...], axis=1, keepdims=True)
    pk_row = jnp.sum(pk_ref[...], axis=1, keepdims=True)
    msum = jnp.sum(rs_row * maskf)
    g = jnp.sum(pk_row * maskf)
    kl_ref[...] = (-_EPS * msum + (_EPS - _CONF) * g).reshape(1, 1)
    nll_ref[...] = (-g).reshape(1, 1)


@jax.jit
def kernel(x, target):
    t32 = target.astype(jnp.int32)
    rs_flat, pk_flat = _sc_colsum(x, t32)
    t2d = t32.reshape(_N, 1)
    kl_tc, nll_tc = pl.pallas_call(
        _loss_block,
        grid=(_N // _BR, _CT // _BC),
        in_specs=[
            pl.BlockSpec((_BR, _BC), lambda i, j: (i, j)),
            pl.BlockSpec((_BR, 1), lambda i, j: (i, 0)),
        ],
        out_specs=[
            pl.BlockSpec((1, 1), lambda i, j: (0, 0)),
            pl.BlockSpec((1, 1), lambda i, j: (0, 0)),
        ],
        out_shape=[
            jax.ShapeDtypeStruct((1, 1), jnp.float32),
            jax.ShapeDtypeStruct((1, 1), jnp.float32),
        ],
    )(x, t2d)
    kl_sc, nll_sc = pl.pallas_call(
        _sc_combine,
        out_shape=[
            jax.ShapeDtypeStruct((1, 1), jnp.float32),
            jax.ShapeDtypeStruct((1, 1), jnp.float32),
        ],
    )(rs_flat.reshape(_N, 16), pk_flat.reshape(_N, 16), t2d)
    kl = kl_tc[0, 0] + kl_sc[0, 0]
    nll = nll_tc[0, 0] + nll_sc[0, 0]
    return (kl, nll)


# speedup vs baseline: 2.5453x; 1.0735x over previous
import functools

import jax
import jax.numpy as jnp
import numpy as np
from jax import lax
from jax.experimental import pallas as pl
from jax.experimental.pallas import tpu as pltpu
from jax.experimental.pallas import tpu_sc as plsc

_SIZE = 32000
_N = 2048
_PAD = 0
_SMOOTH = 0.1
_CONF = 1.0 - _SMOOTH
_EPS = _SMOOTH / (_SIZE - 2)
_C1 = np.float32(_EPS * np.log(_EPS) * (_SIZE - 2) + _CONF * np.log(_CONF))

_CT = 19712
_BR = 256
_BC = 9856

_SC_CHUNK = 4096
_NC = 2
_NS = 16
_NW = _NC * _NS
_BPW = _N // _NW
_NCHUNK = (_SIZE - _CT) // _SC_CHUNK


def _bcast_lane(vec, lane):
    idx = jnp.full((16, 1), lane, jnp.int32)
    dn = lax.GatherDimensionNumbers(
        offset_dims=(), collapsed_slice_dims=(0,), start_index_map=(0,))
    return lax.gather(vec, idx, dn, (1,),
                      mode=lax.GatherScatterMode.PROMISE_IN_BOUNDS)


@functools.partial(
    pl.kernel,
    out_type=[
        jax.ShapeDtypeStruct((_NW * 16,), jnp.float32),
        jax.ShapeDtypeStruct((_NW * 16,), jnp.float32),
    ],
    mesh=plsc.VectorSubcoreMesh(core_axis_name="c", subcore_axis_name="s"),
    scratch_types=[
        pltpu.VMEM((_BPW,), jnp.int32),
        pltpu.VMEM((8, _SC_CHUNK), jnp.float32),
        pltpu.VMEM((8, _SC_CHUNK), jnp.float32),
        pltpu.VMEM((16,), jnp.float32),
        pltpu.VMEM((16,), jnp.float32),
        pltpu.SemaphoreType.DMA,
    ],
)
def _sc_colsum(x_hbm, tgt_hbm, rs_hbm, pk_hbm, tgt_v, buf0_v, buf1_v,
               rs_v, pk_v, sem):
    cid = lax.axis_index("c")
    sid = lax.axis_index("s")
    wid = sid * _NC + cid
    base = wid * _BPW
    lanes = lax.iota(jnp.int32, 16)
    pltpu.sync_copy(tgt_hbm.at[pl.ds(base, _BPW)], tgt_v)
    bufs = [buf0_v, buf1_v]
    ngroup = _BPW // 8
    steps = [(g, ch) for g in range(ngroup) for ch in range(_NCHUNK)]

    def start(step):
        g, ch = steps[step]
        return pltpu.async_copy(
            x_hbm.at[pl.ds(base + g * 8, 8),
                     pl.ds(_CT + ch * _SC_CHUNK, _SC_CHUNK)],
            bufs[step % 2], sem)

    pending = start(0)
    acc_rs = [jnp.zeros((16,), jnp.float32) for _ in range(8)]
    acc_pk = [jnp.zeros((16,), jnp.float32) for _ in range(8)]
    tot_rs = jnp.zeros((16,), jnp.float32)
    tot_pk = jnp.zeros((16,), jnp.float32)
    for step, (g, ch) in enumerate(steps):
        pending.wait()
        if step + 1 < len(steps):
            pending = start(step + 1)
        buf_v = bufs[step % 2]
        col0 = _CT + ch * _SC_CHUNK
        tvec = tgt_v[pl.ds((g // 2) * 16, 16)]
        tb = [_bcast_lane(tvec, 8 * (g % 2) + r) for r in range(8)]

        def body(i, carry, col0=col0, tb=tb, buf_v=buf_v):
            rs, pk = carry
            cvec = col0 + i * 16 + lanes
            rs2, pk2 = [], []
            for r in range(8):
                v = buf_v[r, pl.ds(i * 16, 16)]
                rs2.append(rs[r] + v)
                pk2.append(pk[r] + jnp.where(cvec == tb[r], v, 0.0))
            return rs2, pk2

        acc_rs, acc_pk = lax.fori_loop(
            0, _SC_CHUNK // 16, body, (acc_rs, acc_pk))
        if ch == _NCHUNK - 1:
            for r in range(8):
                mf = jnp.where(tb[r] != _PAD, 1.0, 0.0)
                tot_rs = tot_rs + acc_rs[r] * mf
                tot_pk = tot_pk + acc_pk[r] * mf
            acc_rs = [jnp.zeros((16,), jnp.float32) for _ in range(8)]
            acc_pk = [jnp.zeros((16,), jnp.float32) for _ in range(8)]
    rs_v[...] = tot_rs
    pk_v[...] = tot_pk
    pltpu.sync_copy(rs_v, rs_hbm.at[pl.ds(wid * 16, 16)])
    pltpu.sync_copy(pk_v, pk_hbm.at[pl.ds(wid * 16, 16)])


def _loss_block(x_ref, t_ref, kl_ref, nll_ref):
    i = pl.program_id(0)
    j = pl.program_id(1)
    xb = x_ref[...]
    tb = t_ref[...]
    maskf = (tb != _PAD).astype(jnp.float32)

    rowsum = jnp.sum(xb, axis=1, keepdims=True)
    colid = j * _BC + lax.broadcasted_iota(jnp.int32, (_BR, _BC), 1)
    rowg = jnp.sum(jnp.where(colid == tb, xb, 0.0), axis=1, keepdims=True)

    msum = jnp.sum(rowsum * maskf)
    g = jnp.sum(rowg * maskf)

    @pl.when((i == 0) & (j == 0))
    def _():
        kl_ref[...] = jnp.zeros((1, 1), jnp.float32)
        nll_ref[...] = jnp.zeros((1, 1), jnp.float32)

    @pl.when(j == 0)
    def _():
        extra = _EPS * jnp.sum(xb[:, 0:1] * maskf) + _C1 * jnp.sum(maskf)
        kl_ref[...] += extra.reshape(1, 1)

    kl_ref[...] += (-_EPS * msum + (_EPS - _CONF) * g).reshape(1, 1)
    nll_ref[...] += (-g).reshape(1, 1)


def _sc_combine(rs_ref, pk_ref, kl_ref, nll_ref):
    msum = jnp.sum(rs_ref[...])
    g = jnp.sum(pk_ref[...])
    kl_ref[...] = (-_EPS * msum + (_EPS - _CONF) * g).reshape(1, 1)
    nll_ref[...] = (-g).reshape(1, 1)


@jax.jit
def kernel(x, target):
    t32 = target.astype(jnp.int32)
    rs_flat, pk_flat = _sc_colsum(x, t32)
    t2d = t32.reshape(_N, 1)
    kl_tc, nll_tc = pl.pallas_call(
        _loss_block,
        grid=(_N // _BR, _CT // _BC),
        in_specs=[
            pl.BlockSpec((_BR, _BC), lambda i, j: (i, j)),
            pl.BlockSpec((_BR, 1), lambda i, j: (i, 0)),
        ],
        out_specs=[
            pl.BlockSpec((1, 1), lambda i, j: (0, 0)),
            pl.BlockSpec((1, 1), lambda i, j: (0, 0)),
        ],
        out_shape=[
            jax.ShapeDtypeStruct((1, 1), jnp.float32),
            jax.ShapeDtypeStruct((1, 1), jnp.float32),
        ],
    )(x, t2d)
    kl_sc, nll_sc = pl.pallas_call(
        _sc_combine,
        out_shape=[
            jax.ShapeDtypeStruct((1, 1), jnp.float32),
            jax.ShapeDtypeStruct((1, 1), jnp.float32),
        ],
    )(rs_flat.reshape(_NW, 16), pk_flat.reshape(_NW, 16))
    kl = kl_tc[0, 0] + kl_sc[0, 0]
    nll = nll_tc[0, 0] + nll_sc[0, 0]
    return (kl, nll)


# speedup vs baseline: 3.3610x; 1.3205x over previous
import jax
import jax.numpy as jnp
import numpy as np
from jax import lax
from jax.experimental import pallas as pl

_SIZE = 32000
_N = 2048
_PAD = 0
_SMOOTH = 0.1
_CONF = 1.0 - _SMOOTH
_EPS = _SMOOTH / (_SIZE - 2)
_C1 = np.float32(_EPS * np.log(_EPS) * (_SIZE - 2) + _CONF * np.log(_CONF))

_BR = 256
_BC = 16000


def _loss_block(x_ref, t_ref, kl_ref, nll_ref):
    i = pl.program_id(0)
    j = pl.program_id(1)
    xb = x_ref[...]
    tb = t_ref[...]
    maskf = (tb != _PAD).astype(jnp.float32)

    rowsum = jnp.sum(xb, axis=1, keepdims=True)
    colid = j * _BC + lax.broadcasted_iota(jnp.int32, (_BR, _BC), 1)
    rowg = jnp.sum(jnp.where(colid == tb, xb, 0.0), axis=1, keepdims=True)

    msum = jnp.sum(rowsum * maskf)
    g = jnp.sum(rowg * maskf)

    @pl.when((i == 0) & (j == 0))
    def _():
        kl_ref[...] = jnp.zeros((1, 1), jnp.float32)
        nll_ref[...] = jnp.zeros((1, 1), jnp.float32)

    @pl.when(j == 0)
    def _():
        extra = _EPS * jnp.sum(xb[:, 0:1] * maskf) + _C1 * jnp.sum(maskf)
        kl_ref[...] += extra.reshape(1, 1)

    kl_ref[...] += (-_EPS * msum + (_EPS - _CONF) * g).reshape(1, 1)
    nll_ref[...] += (-g).reshape(1, 1)


@jax.jit
def kernel(x, target):
    t2d = target.astype(jnp.int32).reshape(_N, 1)
    kl, nll = pl.pallas_call(
        _loss_block,
        grid=(_N // _BR, _SIZE // _BC),
        in_specs=[
            pl.BlockSpec((_BR, _BC), lambda i, j: (i, j)),
            pl.BlockSpec((_BR, 1), lambda i, j: (i, 0)),
        ],
        out_specs=[
            pl.BlockSpec((1, 1), lambda i, j: (0, 0)),
            pl.BlockSpec((1, 1), lambda i, j: (0, 0)),
        ],
        out_shape=[
            jax.ShapeDtypeStruct((1, 1), jnp.float32),
            jax.ShapeDtypeStruct((1, 1), jnp.float32),
        ],
    )(x, t2d)
    return (kl[0, 0], nll[0, 0])


# speedup vs baseline: 3.3821x; 1.0063x over previous
import jax
import jax.numpy as jnp
import numpy as np
from jax import lax
from jax.experimental import pallas as pl

_SIZE = 32000
_N = 2048
_PAD = 0
_SMOOTH = 0.1
_CONF = 1.0 - _SMOOTH
_EPS = _SMOOTH / (_SIZE - 2)
_C1 = np.float32(_EPS * np.log(_EPS) * (_SIZE - 2) + _CONF * np.log(_CONF))

_BR = 512
_BC = 6400


def _loss_block(x_ref, t_ref, kl_ref, nll_ref):
    i = pl.program_id(0)
    j = pl.program_id(1)
    xb = x_ref[...]
    tb = t_ref[...]
    maskf = (tb != _PAD).astype(jnp.float32)

    rowsum = jnp.sum(xb, axis=1, keepdims=True)
    colid = j * _BC + lax.broadcasted_iota(jnp.int32, (_BR, _BC), 1)
    rowg = jnp.sum(jnp.where(colid == tb, xb, 0.0), axis=1, keepdims=True)

    msum = jnp.sum(rowsum * maskf)
    g = jnp.sum(rowg * maskf)

    @pl.when((i == 0) & (j == 0))
    def _():
        kl_ref[...] = jnp.zeros((1, 1), jnp.float32)
        nll_ref[...] = jnp.zeros((1, 1), jnp.float32)

    @pl.when(j == 0)
    def _():
        extra = _EPS * jnp.sum(xb[:, 0:1] * maskf) + _C1 * jnp.sum(maskf)
        kl_ref[...] += extra.reshape(1, 1)

    kl_ref[...] += (-_EPS * msum + (_EPS - _CONF) * g).reshape(1, 1)
    nll_ref[...] += (-g).reshape(1, 1)


@jax.jit
def kernel(x, target):
    t2d = target.astype(jnp.int32).reshape(_N, 1)
    kl, nll = pl.pallas_call(
        _loss_block,
        grid=(_N // _BR, _SIZE // _BC),
        in_specs=[
            pl.BlockSpec((_BR, _BC), lambda i, j: (i, j)),
            pl.BlockSpec((_BR, 1), lambda i, j: (i, 0)),
        ],
        out_specs=[
            pl.BlockSpec((1, 1), lambda i, j: (0, 0)),
            pl.BlockSpec((1, 1), lambda i, j: (0, 0)),
        ],
        out_shape=[
            jax.ShapeDtypeStruct((1, 1), jnp.float32),
            jax.ShapeDtypeStruct((1, 1), jnp.float32),
        ],
    )(x, t2d)
    return (kl[0, 0], nll[0, 0])
